# baseline (speedup 1.0000x reference)
import jax
import jax.numpy as jnp
from jax import lax
from jax.experimental import pallas as pl
from jax.experimental.pallas import tpu as pltpu
from jax.experimental.pallas import tpu_sc as plsc

_NC, _NS = 2, 16
_NW = _NC * _NS
_D = 128
_B = 4096 * 200
_ROWS = _B // _D
_RPW = _ROWS // _NW


_NB = 5


def _body(x_hbm, emb_hbm, out_hbm, idx_v, b0, b1, b2, b3, b4, s0, s1, s2, s3, s4):
    bufs = (b0, b1, b2, b3, b4)
    sems = (s0, s1, s2, s3, s4)
    wid = lax.axis_index("s") * _NC + lax.axis_index("c")
    r0 = wid * _RPW
    pltpu.sync_copy(x_hbm.at[pl.ds(r0, _RPW)], idx_v)
    for b in range(_NB):
        pltpu.async_copy(emb_hbm.at[idx_v.at[b]], bufs[b], sems[b])

    def outer(i, carry):
        j = i * _NB
        for b in range(_NB):
            t = j + b
            pltpu.make_async_copy(emb_hbm.at[idx_v.at[t]], bufs[b], sems[b]).wait()
            pltpu.sync_copy(bufs[b], out_hbm.at[pl.ds((r0 + t) * _D, _D)])
            tn = jnp.minimum(t + _NB, _RPW - 1)
            pltpu.async_copy(emb_hbm.at[idx_v.at[tn]], bufs[b], sems[b])
        return carry

    lax.fori_loop(0, _RPW // _NB, outer, 0)
    for b in range(_NB):
        pltpu.make_async_copy(emb_hbm.at[idx_v.at[0]], bufs[b], sems[b]).wait()


def kernel(x, emb):
    xf = x.reshape(_ROWS, _D)
    mesh = plsc.VectorSubcoreMesh(core_axis_name="c", subcore_axis_name="s")
    out = pl.kernel(
        _body,
        out_type=jax.ShapeDtypeStruct((_B, _D), jnp.float32),
        mesh=mesh,
        scratch_types=[
            pltpu.VMEM((_RPW, _D), jnp.int32),
        ] + [pltpu.VMEM((_D, _D), jnp.float32)] * _NB
          + [pltpu.SemaphoreType.DMA] * _NB,
    )(xf, emb)
    return out.reshape(x.shape[0], x.shape[1], _D)


# speedup vs baseline: 1.0032x; 1.0032x over previous
import jax
import jax.numpy as jnp
from jax import lax
from jax.experimental import pallas as pl
from jax.experimental.pallas import tpu as pltpu
from jax.experimental.pallas import tpu_sc as plsc

_NC, _NS = 2, 16
_NW = _NC * _NS
_D = 128
_B = 4096 * 200
_ROWS = _B // _D
_RPW = _ROWS // _NW


_NB = 6
_LAG = 2


def _body(x_hbm, emb_hbm, out_hbm, idx_v,
          b0, b1, b2, b3, b4, b5,
          g0, g1, g2, g3, g4, g5,
          w0, w1, w2, w3, w4, w5):
    bufs = (b0, b1, b2, b3, b4, b5)
    gs = (g0, g1, g2, g3, g4, g5)
    ws = (w0, w1, w2, w3, w4, w5)
    wid = lax.axis_index("s") * _NC + lax.axis_index("c")
    r0 = wid * _RPW
    pltpu.sync_copy(x_hbm.at[pl.ds(r0, _RPW)], idx_v)

    ahead = _NB - _LAG

    def wait_gather(t, b):
        pltpu.make_async_copy(emb_hbm.at[idx_v.at[t]], bufs[b], gs[b]).wait()

    def issue_write(t, b):
        pltpu.async_copy(bufs[b], out_hbm.at[pl.ds((r0 + t) * _D, _D)], ws[b])

    def wait_write(b):
        pltpu.make_async_copy(bufs[b], out_hbm.at[pl.ds(r0 * _D, _D)], ws[b]).wait()

    for t in range(ahead):
        pltpu.async_copy(emb_hbm.at[idx_v.at[t]], bufs[t % _NB], gs[t % _NB])
    for s in range(_LAG):
        wait_gather(s, s % _NB)
        issue_write(s, s % _NB)
        bn = (s + ahead) % _NB
        pltpu.async_copy(emb_hbm.at[idx_v.at[s + ahead]], bufs[bn], gs[bn])

    def step(i, carry):
        for k in range(_NB):
            s = _LAG + i * _NB + k
            b = (_LAG + k) % _NB
            wait_gather(s, b)
            issue_write(s, b)
            bn = k
            wait_write(bn)
            tn = jnp.minimum(s + ahead, _RPW - 1)
            pltpu.async_copy(emb_hbm.at[idx_v.at[tn]], bufs[bn], gs[bn])
        return carry

    lax.fori_loop(0, (_RPW - _LAG) // _NB, step, 0)

    for s in range(_RPW, _RPW + ahead):
        wait_gather(0, s % _NB)
    for s in range(_RPW - _LAG, _RPW):
        wait_write(s % _NB)


def kernel(x, emb):
    xf = x.reshape(_ROWS, _D)
    mesh = plsc.VectorSubcoreMesh(core_axis_name="c", subcore_axis_name="s")
    out = pl.kernel(
        _body,
        out_type=jax.ShapeDtypeStruct((_B, _D), jnp.float32),
        mesh=mesh,
        scratch_types=[
            pltpu.VMEM((_RPW, _D), jnp.int32),
        ] + [pltpu.VMEM((_D, _D), jnp.float32)] * _NB
          + [pltpu.SemaphoreType.DMA] * (2 * _NB),
    )(xf, emb)
    return out.reshape(x.shape[0], x.shape[1], _D)


# speedup vs baseline: 1.0505x; 1.0471x over previous
import jax
import jax.numpy as jnp
from jax import lax
from jax.experimental import pallas as pl
from jax.experimental.pallas import tpu as pltpu
from jax.experimental.pallas import tpu_sc as plsc

_NC, _NS = 2, 16
_NW = _NC * _NS
_D = 128
_B = 4096 * 200
_ROWS = _B // _D
_RPW = _ROWS // _NW

_NB = 4
_AHEAD = 4


def _body(x_hbm, emb_hbm, out_hbm, idx_v, spm_all,
          b0, b1, b2, b3,
          g0, g1, g2, g3,
          a0, a1, w0, w1):
    bufs = (b0, b1, b2, b3)
    gs = (g0, g1, g2, g3)
    asem = (a0, a1)
    bsem = (w0, w1)
    wid = lax.axis_index("s") * _NC + lax.axis_index("c")
    spm = spm_all.at[lax.axis_index("s")]
    r0 = wid * _RPW
    pltpu.sync_copy(x_hbm.at[pl.ds(r0, _RPW)], idx_v)

    def wait_gather(t, b):
        pltpu.make_async_copy(emb_hbm.at[idx_v.at[t]], bufs[b], gs[b]).wait()

    def step(t, b, q, first):
        wait_gather(t, b)
        if not first:
            pltpu.make_async_copy(spm.at[q], out_hbm.at[pl.ds(r0 * _D, _D)],
                                  bsem[q]).wait()
        pltpu.async_copy(bufs[b], spm.at[q], asem[q]).wait()
        pltpu.async_copy(spm.at[q], out_hbm.at[pl.ds((r0 + t) * _D, _D)],
                         bsem[q])

    for t in range(_AHEAD):
        pltpu.async_copy(emb_hbm.at[idx_v.at[t]], bufs[t % _NB], gs[t % _NB])
    for t in range(_NB):
        step(t, t % _NB, t % 2, first=(t < 2))
        bn = (t + _AHEAD) % _NB
        pltpu.async_copy(emb_hbm.at[idx_v.at[t + _AHEAD]], bufs[bn], gs[bn])

    def outer(i, carry):
        for k in range(_NB):
            t = _NB + i * _NB + k
            step(t, k, k % 2, first=False)
            bn = (k + _AHEAD) % _NB
            tn = jnp.minimum(t + _AHEAD, _RPW - 1)
            pltpu.async_copy(emb_hbm.at[idx_v.at[tn]], bufs[bn], gs[bn])
        return carry

    lax.fori_loop(0, (_RPW - _NB) // _NB, outer, 0)

    for t in range(_RPW, _RPW + _AHEAD):
        wait_gather(0, t % _NB)
    for q in range(2):
        pltpu.make_async_copy(spm.at[q], out_hbm.at[pl.ds(r0 * _D, _D)],
                              bsem[q]).wait()


def kernel(x, emb):
    xf = x.reshape(_ROWS, _D)
    mesh = plsc.VectorSubcoreMesh(core_axis_name="c", subcore_axis_name="s")
    out = pl.kernel(
        _body,
        out_type=jax.ShapeDtypeStruct((_B, _D), jnp.float32),
        mesh=mesh,
        scratch_types=[
            pltpu.VMEM((_RPW, _D), jnp.int32),
            pltpu.VMEM_SHARED((_NS, 2, _D, _D), jnp.float32),
        ] + [pltpu.VMEM((_D, _D), jnp.float32)] * _NB
          + [pltpu.SemaphoreType.DMA] * (_NB + 4),
    )(xf, emb)
    return out.reshape(x.shape[0], x.shape[1], _D)
